# baseline (speedup 1.0000x reference)
import functools

import jax
import jax.numpy as jnp
from jax.experimental import pallas as pl
from jax.experimental.pallas import tpu as pltpu

H = 8


def _body(hid_ref, s0_ref, wp_ref, bb_ref, ba_ref, bd_ref, gwt_ref, wot_ref,
          nw_ref, out_ref, sout_ref,
          S_scr, A_scr, Bt_scr, V_scr, ctx_scr, K2_scr, Q2_scr,
          *, C, BPC, SD, D, HID):
    HB = BPC * H
    NHD = BPC * SD
    n = pl.program_id(0)

    @pl.when(n == 0)
    def _():
        S_scr[...] = s0_ref[...]

    for b in range(BPC):
        h_b = hid_ref[b]
        p = jnp.dot(h_b.astype(jnp.bfloat16), wp_ref[...],
                    preferred_element_type=jnp.float32)
        k = p[:, 0:SD]
        v = p[:, SD:2 * SD]
        q = p[:, 2 * SD:3 * SD]
        beta = jax.nn.sigmoid(p[:, 3 * SD:4 * SD] + bb_ref[...])
        alpha = jax.nn.sigmoid(p[:, 4 * SD:5 * SD] + ba_ref[...])
        ss = jnp.dot((k * k).astype(jnp.bfloat16), bd_ref[...],
                     preferred_element_type=jnp.float32)
        k = k / jnp.maximum(jnp.sqrt(ss), 1e-12)
        A_scr[:, :, b * SD:(b + 1) * SD] = alpha[:, None, :]
        Bt_scr[:, :, b * SD:(b + 1) * SD] = beta[:, None, :]
        V_scr[:, :, b * SD:(b + 1) * SD] = (beta * v)[:, None, :]
        for h in range(H):
            K2_scr[:, b * H + h, :] = k[:, h * D:(h + 1) * D]
            Q2_scr[:, b * H + h, :] = q[:, h * D:(h + 1) * D]

    lane = jax.lax.broadcasted_iota(jnp.int32, (HB, NHD), 1)
    sub = jax.lax.broadcasted_iota(jnp.int32, (HB, NHD), 0)
    Mb = (lane // D) == sub

    def step(t, S):
        a_row = A_scr[pl.ds(t, 1)].reshape(1, NHD)
        b_row = Bt_scr[pl.ds(t, 1)].reshape(1, NHD)
        v_row = V_scr[pl.ds(t, 1)].reshape(1, NHD)
        k_mat = K2_scr[pl.ds(t, 1)].reshape(HB, D)
        q_mat = Q2_scr[pl.ds(t, 1)].reshape(HB, D)
        Sd = S * a_row
        pred32 = jnp.dot(k_mat, Sd, preferred_element_type=jnp.float32)
        pred_row = jnp.sum(jnp.where(Mb, pred32, 0.0), axis=0, keepdims=True)
        u_row = v_row - b_row * pred_row
        U = jnp.where(Mb, jnp.broadcast_to(u_row, (HB, NHD)), 0.0)
        dS = jax.lax.dot_general(k_mat, U, (((0,), (0,)), ((), ())),
                                 preferred_element_type=jnp.float32)
        S2 = Sd + dS
        r32 = jnp.dot(q_mat, S2, preferred_element_type=jnp.float32)
        r_row = jnp.sum(jnp.where(Mb, r32, 0.0), axis=0, keepdims=True)
        ctx_scr[pl.ds(t, 1)] = r_row.reshape(1, 1, NHD)
        return S2

    S = jax.lax.fori_loop(0, C, step, S_scr[...], unroll=8)
    S_scr[...] = S
    sout_ref[...] = S

    ctx2 = ctx_scr[...].reshape(C, NHD)
    for b in range(BPC):
        cb = ctx2[:, b * SD:(b + 1) * SD]
        msq = jnp.sum(cb * cb, axis=-1, keepdims=True) * (1.0 / SD)
        normed = cb * jax.lax.rsqrt(msq + 1e-6) * nw_ref[...]
        g = jnp.dot(hid_ref[b].astype(jnp.bfloat16), gwt_ref[...],
                    preferred_element_type=jnp.float32)
        act = normed * (g * jax.nn.sigmoid(g))
        out_ref[b] = jnp.dot(act.astype(jnp.bfloat16), wot_ref[...],
                             preferred_element_type=jnp.float32)


def kernel(hidden_states, state, W_k, W_v, W_q, W_beta, b_beta, W_alpha,
           b_alpha, W_out, gate_W, norm_w):
    B, T, HID = hidden_states.shape
    SD = W_k.shape[0]
    D = SD // H
    C = 256 if T % 256 == 0 else T
    NCH = T // C
    BPC = B
    NHD = BPC * SD
    HB = BPC * H

    f32 = jnp.float32
    bf16 = jnp.bfloat16
    Wp = jnp.concatenate([W_k, W_v, W_q, W_beta, W_alpha],
                         axis=0).T.astype(bf16)
    bb = b_beta[None].astype(f32)
    ba = b_alpha[None].astype(f32)
    nw = norm_w[None].astype(f32)
    gwt = gate_W.T.astype(bf16)
    wot = W_out.T.astype(bf16)
    S0T = state.transpose(3, 0, 1, 2).reshape(D, B * SD)
    ii = jnp.arange(SD, dtype=jnp.int32)
    BD = ((ii[:, None] // D) == (ii[None, :] // D)).astype(bf16)

    body = functools.partial(_body, C=C, BPC=BPC, SD=SD, D=D, HID=HID)
    out, soutT = pl.pallas_call(
        body,
        grid=(NCH,),
        in_specs=[
            pl.BlockSpec((BPC, C, HID), lambda n: (0, n, 0)),
            pl.BlockSpec((D, NHD), lambda n: (0, 0)),
            pl.BlockSpec((HID, 5 * SD), lambda n: (0, 0)),
            pl.BlockSpec((1, SD), lambda n: (0, 0)),
            pl.BlockSpec((1, SD), lambda n: (0, 0)),
            pl.BlockSpec((SD, SD), lambda n: (0, 0)),
            pl.BlockSpec((HID, SD), lambda n: (0, 0)),
            pl.BlockSpec((SD, HID), lambda n: (0, 0)),
            pl.BlockSpec((1, SD), lambda n: (0, 0)),
        ],
        out_specs=[
            pl.BlockSpec((BPC, C, HID), lambda n: (0, n, 0)),
            pl.BlockSpec((D, NHD), lambda n: (0, 0)),
        ],
        out_shape=[
            jax.ShapeDtypeStruct((B, T, HID), f32),
            jax.ShapeDtypeStruct((D, B * SD), f32),
        ],
        scratch_shapes=[
            pltpu.VMEM((D, NHD), f32),
            pltpu.VMEM((C, 1, NHD), f32),
            pltpu.VMEM((C, 1, NHD), f32),
            pltpu.VMEM((C, 1, NHD), f32),
            pltpu.VMEM((C, 1, NHD), f32),
            pltpu.VMEM((C, HB, D), f32),
            pltpu.VMEM((C, HB, D), f32),
        ],
        compiler_params=pltpu.CompilerParams(
            dimension_semantics=("arbitrary",),
            vmem_limit_bytes=100 * 1024 * 1024,
        ),
        name="gated_delta_state_fused",
    )(hidden_states, S0T, Wp, bb, ba, BD, gwt, wot, nw)

    S_final = soutT.reshape(D, B, H, D).transpose(1, 2, 3, 0)
    return out, S_final


# speedup vs baseline: 1.0615x; 1.0615x over previous
import functools

import jax
import jax.numpy as jnp
from jax.experimental import pallas as pl
from jax.experimental.pallas import tpu as pltpu

H = 8


def _body(hid_ref, s0_ref, wp_ref, bb_ref, ba_ref, bd_ref, gwt_ref, wot_ref,
          nw_ref, out_ref, sout_ref,
          S_scr, A_scr, Bt_scr, V_scr, ctx_scr, K2_scr, Q2_scr,
          *, C, BPC, SD, D, HID):
    HB = BPC * H
    NHD = BPC * SD
    n = pl.program_id(0)

    @pl.when(n == 0)
    def _():
        S_scr[...] = s0_ref[...]

    for b in range(BPC):
        h_b = hid_ref[b]
        p = jnp.dot(h_b.astype(jnp.bfloat16), wp_ref[...],
                    preferred_element_type=jnp.float32)
        k = p[:, 0:SD]
        v = p[:, SD:2 * SD]
        q = p[:, 2 * SD:3 * SD]
        beta = jax.nn.sigmoid(p[:, 3 * SD:4 * SD] + bb_ref[...])
        alpha = jax.nn.sigmoid(p[:, 4 * SD:5 * SD] + ba_ref[...])
        ss = jnp.dot((k * k).astype(jnp.bfloat16), bd_ref[...],
                     preferred_element_type=jnp.float32)
        k = k / jnp.maximum(jnp.sqrt(ss), 1e-12)
        A_scr[:, :, b * SD:(b + 1) * SD] = alpha[:, None, :]
        Bt_scr[:, :, b * SD:(b + 1) * SD] = beta[:, None, :]
        V_scr[:, :, b * SD:(b + 1) * SD] = (beta * v)[:, None, :]
        for h in range(H):
            K2_scr[:, b * H + h, :] = k[:, h * D:(h + 1) * D]
            Q2_scr[:, b * H + h, :] = q[:, h * D:(h + 1) * D]

    lane = jax.lax.broadcasted_iota(jnp.int32, (HB, NHD), 1)
    sub = jax.lax.broadcasted_iota(jnp.int32, (HB, NHD), 0)
    Mb = (lane // D) == sub

    def step(t, S):
        a_row = A_scr[pl.ds(t, 1)].reshape(1, NHD)
        b_row = Bt_scr[pl.ds(t, 1)].reshape(1, NHD)
        v_row = V_scr[pl.ds(t, 1)].reshape(1, NHD)
        k_mat = K2_scr[pl.ds(t, 1)].reshape(HB, D)
        q_mat = Q2_scr[pl.ds(t, 1)].reshape(HB, D)
        pred32 = jnp.dot(k_mat, S, preferred_element_type=jnp.float32)
        Sd = S * a_row
        pred_row = a_row * jnp.sum(jnp.where(Mb, pred32, 0.0), axis=0,
                                   keepdims=True)
        u_row = v_row - b_row * pred_row
        U = jnp.where(Mb, jnp.broadcast_to(u_row, (HB, NHD)), 0.0)
        dS = jax.lax.dot_general(k_mat, U, (((0,), (0,)), ((), ())),
                                 preferred_element_type=jnp.float32)
        S2 = Sd + dS
        r32 = jnp.dot(q_mat, S2, preferred_element_type=jnp.float32)
        r_row = jnp.sum(jnp.where(Mb, r32, 0.0), axis=0, keepdims=True)
        ctx_scr[pl.ds(t, 1)] = r_row.reshape(1, 1, NHD)
        return S2

    S = jax.lax.fori_loop(0, C, step, S_scr[...], unroll=8)
    S_scr[...] = S
    sout_ref[...] = S

    ctx2 = ctx_scr[...].reshape(C, NHD)
    for b in range(BPC):
        cb = ctx2[:, b * SD:(b + 1) * SD]
        msq = jnp.sum(cb * cb, axis=-1, keepdims=True) * (1.0 / SD)
        normed = cb * jax.lax.rsqrt(msq + 1e-6) * nw_ref[...]
        g = jnp.dot(hid_ref[b].astype(jnp.bfloat16), gwt_ref[...],
                    preferred_element_type=jnp.float32)
        act = normed * (g * jax.nn.sigmoid(g))
        out_ref[b] = jnp.dot(act.astype(jnp.bfloat16), wot_ref[...],
                             preferred_element_type=jnp.float32)


def kernel(hidden_states, state, W_k, W_v, W_q, W_beta, b_beta, W_alpha,
           b_alpha, W_out, gate_W, norm_w):
    B, T, HID = hidden_states.shape
    SD = W_k.shape[0]
    D = SD // H
    C = 128 if T % 128 == 0 else T
    NCH = T // C
    BPC = B
    NHD = BPC * SD
    HB = BPC * H

    f32 = jnp.float32
    bf16 = jnp.bfloat16
    Wp = jnp.concatenate([W_k, W_v, W_q, W_beta, W_alpha],
                         axis=0).T.astype(bf16)
    bb = b_beta[None].astype(f32)
    ba = b_alpha[None].astype(f32)
    nw = norm_w[None].astype(f32)
    gwt = gate_W.T.astype(bf16)
    wot = W_out.T.astype(bf16)
    S0T = state.transpose(3, 0, 1, 2).reshape(D, B * SD)
    ii = jnp.arange(SD, dtype=jnp.int32)
    BD = ((ii[:, None] // D) == (ii[None, :] // D)).astype(bf16)

    body = functools.partial(_body, C=C, BPC=BPC, SD=SD, D=D, HID=HID)
    out, soutT = pl.pallas_call(
        body,
        grid=(NCH,),
        in_specs=[
            pl.BlockSpec((BPC, C, HID), lambda n: (0, n, 0)),
            pl.BlockSpec((D, NHD), lambda n: (0, 0)),
            pl.BlockSpec((HID, 5 * SD), lambda n: (0, 0)),
            pl.BlockSpec((1, SD), lambda n: (0, 0)),
            pl.BlockSpec((1, SD), lambda n: (0, 0)),
            pl.BlockSpec((SD, SD), lambda n: (0, 0)),
            pl.BlockSpec((HID, SD), lambda n: (0, 0)),
            pl.BlockSpec((SD, HID), lambda n: (0, 0)),
            pl.BlockSpec((1, SD), lambda n: (0, 0)),
        ],
        out_specs=[
            pl.BlockSpec((BPC, C, HID), lambda n: (0, n, 0)),
            pl.BlockSpec((D, NHD), lambda n: (0, 0)),
        ],
        out_shape=[
            jax.ShapeDtypeStruct((B, T, HID), f32),
            jax.ShapeDtypeStruct((D, B * SD), f32),
        ],
        scratch_shapes=[
            pltpu.VMEM((D, NHD), f32),
            pltpu.VMEM((C, 1, NHD), f32),
            pltpu.VMEM((C, 1, NHD), f32),
            pltpu.VMEM((C, 1, NHD), f32),
            pltpu.VMEM((C, 1, NHD), f32),
            pltpu.VMEM((C, HB, D), f32),
            pltpu.VMEM((C, HB, D), f32),
        ],
        compiler_params=pltpu.CompilerParams(
            dimension_semantics=("arbitrary",),
            vmem_limit_bytes=100 * 1024 * 1024,
        ),
        name="gated_delta_state_fused",
    )(hidden_states, S0T, Wp, bb, ba, BD, gwt, wot, nw)

    S_final = soutT.reshape(D, B, H, D).transpose(1, 2, 3, 0)
    return out, S_final


# speedup vs baseline: 1.0838x; 1.0211x over previous
import functools

import jax
import jax.numpy as jnp
from jax.experimental import pallas as pl
from jax.experimental.pallas import tpu as pltpu

H = 8


def _body(hid_ref, s0_ref, wp_ref, bb_ref, ba_ref, bd_ref, gwt_ref, wot_ref,
          nw_ref, out_ref, sout_ref,
          S_scr, A_scr, Bt_scr, V_scr, ctx_scr, K2_scr, Q2_scr,
          *, C, BPC, SD, D, HID):
    HB = BPC * H
    NHD = BPC * SD
    n = pl.program_id(0)

    @pl.when(n == 0)
    def _():
        S_scr[...] = s0_ref[...]

    for b in range(BPC):
        h_b = hid_ref[b]
        p = jnp.dot(h_b.astype(jnp.bfloat16), wp_ref[...],
                    preferred_element_type=jnp.float32)
        k = p[:, 0:SD]
        v = p[:, SD:2 * SD]
        q = p[:, 2 * SD:3 * SD]
        beta = jax.nn.sigmoid(p[:, 3 * SD:4 * SD] + bb_ref[...])
        alpha = jax.nn.sigmoid(p[:, 4 * SD:5 * SD] + ba_ref[...])
        ss = jnp.dot((k * k).astype(jnp.bfloat16), bd_ref[...],
                     preferred_element_type=jnp.float32)
        k = k / jnp.maximum(jnp.sqrt(ss), 1e-12)
        A_scr[:, :, b * SD:(b + 1) * SD] = alpha[:, None, :]
        Bt_scr[:, :, b * SD:(b + 1) * SD] = beta[:, None, :]
        V_scr[:, :, b * SD:(b + 1) * SD] = (beta * v)[:, None, :]
        for h in range(H):
            K2_scr[:, b * H + h, :] = k[:, h * D:(h + 1) * D]
            Q2_scr[:, b * H + h, :] = q[:, h * D:(h + 1) * D]

    lane = jax.lax.broadcasted_iota(jnp.int32, (HB, NHD), 1)
    sub = jax.lax.broadcasted_iota(jnp.int32, (HB, NHD), 0)
    Mb = (lane // D) == sub

    def step(t, S):
        a_row = A_scr[pl.ds(t, 1)].reshape(1, NHD)
        b_row = Bt_scr[pl.ds(t, 1)].reshape(1, NHD)
        v_row = V_scr[pl.ds(t, 1)].reshape(1, NHD)
        k_mat = K2_scr[pl.ds(t, 1)].reshape(HB, D)
        q_mat = Q2_scr[pl.ds(t, 1)].reshape(HB, D)
        pred32 = jnp.dot(k_mat, S, preferred_element_type=jnp.float32)
        Sd = S * a_row
        pred_row = a_row * jnp.sum(jnp.where(Mb, pred32, 0.0), axis=0,
                                   keepdims=True)
        u_row = v_row - b_row * pred_row
        U = jnp.where(Mb, jnp.broadcast_to(u_row, (HB, NHD)), 0.0)
        dS = jax.lax.dot_general(k_mat, U, (((0,), (0,)), ((), ())),
                                 preferred_element_type=jnp.float32)
        S2 = Sd + dS
        r32 = jnp.dot(q_mat, S2, preferred_element_type=jnp.float32)
        r_row = jnp.sum(jnp.where(Mb, r32, 0.0), axis=0, keepdims=True)
        ctx_scr[pl.ds(t, 1)] = r_row.reshape(1, 1, NHD)
        return S2

    S = jax.lax.fori_loop(0, C, step, S_scr[...], unroll=16)
    S_scr[...] = S
    sout_ref[...] = S

    ctx2 = ctx_scr[...].reshape(C, NHD)
    for b in range(BPC):
        cb = ctx2[:, b * SD:(b + 1) * SD]
        msq = jnp.sum(cb * cb, axis=-1, keepdims=True) * (1.0 / SD)
        normed = cb * jax.lax.rsqrt(msq + 1e-6) * nw_ref[...]
        g = jnp.dot(hid_ref[b].astype(jnp.bfloat16), gwt_ref[...],
                    preferred_element_type=jnp.float32)
        act = normed * (g * jax.nn.sigmoid(g))
        out_ref[b] = jnp.dot(act.astype(jnp.bfloat16), wot_ref[...],
                             preferred_element_type=jnp.float32)


def kernel(hidden_states, state, W_k, W_v, W_q, W_beta, b_beta, W_alpha,
           b_alpha, W_out, gate_W, norm_w):
    B, T, HID = hidden_states.shape
    SD = W_k.shape[0]
    D = SD // H
    C = 128 if T % 128 == 0 else T
    NCH = T // C
    BPC = B
    NHD = BPC * SD
    HB = BPC * H

    f32 = jnp.float32
    bf16 = jnp.bfloat16
    Wp = jnp.concatenate([W_k, W_v, W_q, W_beta, W_alpha],
                         axis=0).T.astype(bf16)
    bb = b_beta[None].astype(f32)
    ba = b_alpha[None].astype(f32)
    nw = norm_w[None].astype(f32)
    gwt = gate_W.T.astype(bf16)
    wot = W_out.T.astype(bf16)
    S0T = state.transpose(3, 0, 1, 2).reshape(D, B * SD)
    ii = jnp.arange(SD, dtype=jnp.int32)
    BD = ((ii[:, None] // D) == (ii[None, :] // D)).astype(bf16)

    body = functools.partial(_body, C=C, BPC=BPC, SD=SD, D=D, HID=HID)
    out, soutT = pl.pallas_call(
        body,
        grid=(NCH,),
        in_specs=[
            pl.BlockSpec((BPC, C, HID), lambda n: (0, n, 0)),
            pl.BlockSpec((D, NHD), lambda n: (0, 0)),
            pl.BlockSpec((HID, 5 * SD), lambda n: (0, 0)),
            pl.BlockSpec((1, SD), lambda n: (0, 0)),
            pl.BlockSpec((1, SD), lambda n: (0, 0)),
            pl.BlockSpec((SD, SD), lambda n: (0, 0)),
            pl.BlockSpec((HID, SD), lambda n: (0, 0)),
            pl.BlockSpec((SD, HID), lambda n: (0, 0)),
            pl.BlockSpec((1, SD), lambda n: (0, 0)),
        ],
        out_specs=[
            pl.BlockSpec((BPC, C, HID), lambda n: (0, n, 0)),
            pl.BlockSpec((D, NHD), lambda n: (0, 0)),
        ],
        out_shape=[
            jax.ShapeDtypeStruct((B, T, HID), f32),
            jax.ShapeDtypeStruct((D, B * SD), f32),
        ],
        scratch_shapes=[
            pltpu.VMEM((D, NHD), f32),
            pltpu.VMEM((C, 1, NHD), f32),
            pltpu.VMEM((C, 1, NHD), f32),
            pltpu.VMEM((C, 1, NHD), f32),
            pltpu.VMEM((C, 1, NHD), f32),
            pltpu.VMEM((C, HB, D), f32),
            pltpu.VMEM((C, HB, D), f32),
        ],
        compiler_params=pltpu.CompilerParams(
            dimension_semantics=("arbitrary",),
            vmem_limit_bytes=100 * 1024 * 1024,
        ),
        name="gated_delta_state_fused",
    )(hidden_states, S0T, Wp, bb, ba, BD, gwt, wot, nw)

    S_final = soutT.reshape(D, B, H, D).transpose(1, 2, 3, 0)
    return out, S_final


# speedup vs baseline: 1.0858x; 1.0018x over previous
import functools

import jax
import jax.numpy as jnp
from jax.experimental import pallas as pl
from jax.experimental.pallas import tpu as pltpu

H = 8


def _body(hid_ref, s0_ref, wp_ref, bb_ref, ba_ref, bd_ref, gwt_ref, wot_ref,
          nw_ref, out_ref, sout_ref,
          S_scr, A_scr, Bt_scr, V_scr, ctx_scr, K2_scr, Q2_scr,
          *, C, BPC, SD, D, HID):
    HB = BPC * H
    NHD = BPC * SD
    n = pl.program_id(0)

    @pl.when(n == 0)
    def _():
        S_scr[...] = s0_ref[...]

    for b in range(BPC):
        h_b = hid_ref[b]
        p = jnp.dot(h_b.astype(jnp.bfloat16), wp_ref[...],
                    preferred_element_type=jnp.float32)
        k = p[:, 0:SD]
        v = p[:, SD:2 * SD]
        q = p[:, 2 * SD:3 * SD]
        beta = jax.nn.sigmoid(p[:, 3 * SD:4 * SD] + bb_ref[...])
        alpha = jax.nn.sigmoid(p[:, 4 * SD:5 * SD] + ba_ref[...])
        ss = jnp.dot((k * k).astype(jnp.bfloat16), bd_ref[...],
                     preferred_element_type=jnp.float32)
        k = k / jnp.maximum(jnp.sqrt(ss), 1e-12)
        A_scr[:, :, b * SD:(b + 1) * SD] = alpha[:, None, :]
        Bt_scr[:, :, b * SD:(b + 1) * SD] = beta[:, None, :]
        V_scr[:, :, b * SD:(b + 1) * SD] = (beta * v)[:, None, :]
        for h in range(H):
            K2_scr[:, b * H + h, :] = k[:, h * D:(h + 1) * D]
            Q2_scr[:, b * H + h, :] = q[:, h * D:(h + 1) * D]

    lane = jax.lax.broadcasted_iota(jnp.int32, (HB, NHD), 1)
    sub = jax.lax.broadcasted_iota(jnp.int32, (HB, NHD), 0)
    Mb = (lane // D) == sub

    def step(t, S):
        a_row = A_scr[pl.ds(t, 1)].reshape(1, NHD)
        b_row = Bt_scr[pl.ds(t, 1)].reshape(1, NHD)
        v_row = V_scr[pl.ds(t, 1)].reshape(1, NHD)
        k_mat = K2_scr[pl.ds(t, 1)].reshape(HB, D)
        q_mat = Q2_scr[pl.ds(t, 1)].reshape(HB, D)
        pred32 = jnp.dot(k_mat, S, preferred_element_type=jnp.float32)
        Sd = S * a_row
        pred_row = a_row * jnp.sum(jnp.where(Mb, pred32, 0.0), axis=0,
                                   keepdims=True)
        u_row = v_row - b_row * pred_row
        U = jnp.where(Mb, jnp.broadcast_to(u_row, (HB, NHD)), 0.0)
        dS = jax.lax.dot_general(k_mat, U, (((0,), (0,)), ((), ())),
                                 preferred_element_type=jnp.float32)
        S2 = Sd + dS
        r32 = jnp.dot(q_mat, S2, preferred_element_type=jnp.float32)
        r_row = jnp.sum(jnp.where(Mb, r32, 0.0), axis=0, keepdims=True)
        ctx_scr[pl.ds(t, 1)] = r_row.reshape(1, 1, NHD)
        return S2

    S = jax.lax.fori_loop(0, C, step, S_scr[...], unroll=32)
    S_scr[...] = S
    sout_ref[...] = S

    ctx2 = ctx_scr[...].reshape(C, NHD)
    for b in range(BPC):
        cb = ctx2[:, b * SD:(b + 1) * SD]
        msq = jnp.sum(cb * cb, axis=-1, keepdims=True) * (1.0 / SD)
        normed = cb * jax.lax.rsqrt(msq + 1e-6) * nw_ref[...]
        g = jnp.dot(hid_ref[b].astype(jnp.bfloat16), gwt_ref[...],
                    preferred_element_type=jnp.float32)
        act = normed * (g * jax.nn.sigmoid(g))
        out_ref[b] = jnp.dot(act.astype(jnp.bfloat16), wot_ref[...],
                             preferred_element_type=jnp.float32)


def kernel(hidden_states, state, W_k, W_v, W_q, W_beta, b_beta, W_alpha,
           b_alpha, W_out, gate_W, norm_w):
    B, T, HID = hidden_states.shape
    SD = W_k.shape[0]
    D = SD // H
    C = 128 if T % 128 == 0 else T
    NCH = T // C
    BPC = B
    NHD = BPC * SD
    HB = BPC * H

    f32 = jnp.float32
    bf16 = jnp.bfloat16
    Wp = jnp.concatenate([W_k, W_v, W_q, W_beta, W_alpha],
                         axis=0).T.astype(bf16)
    bb = b_beta[None].astype(f32)
    ba = b_alpha[None].astype(f32)
    nw = norm_w[None].astype(f32)
    gwt = gate_W.T.astype(bf16)
    wot = W_out.T.astype(bf16)
    S0T = state.transpose(3, 0, 1, 2).reshape(D, B * SD)
    ii = jnp.arange(SD, dtype=jnp.int32)
    BD = ((ii[:, None] // D) == (ii[None, :] // D)).astype(bf16)

    body = functools.partial(_body, C=C, BPC=BPC, SD=SD, D=D, HID=HID)
    out, soutT = pl.pallas_call(
        body,
        grid=(NCH,),
        in_specs=[
            pl.BlockSpec((BPC, C, HID), lambda n: (0, n, 0)),
            pl.BlockSpec((D, NHD), lambda n: (0, 0)),
            pl.BlockSpec((HID, 5 * SD), lambda n: (0, 0)),
            pl.BlockSpec((1, SD), lambda n: (0, 0)),
            pl.BlockSpec((1, SD), lambda n: (0, 0)),
            pl.BlockSpec((SD, SD), lambda n: (0, 0)),
            pl.BlockSpec((HID, SD), lambda n: (0, 0)),
            pl.BlockSpec((SD, HID), lambda n: (0, 0)),
            pl.BlockSpec((1, SD), lambda n: (0, 0)),
        ],
        out_specs=[
            pl.BlockSpec((BPC, C, HID), lambda n: (0, n, 0)),
            pl.BlockSpec((D, NHD), lambda n: (0, 0)),
        ],
        out_shape=[
            jax.ShapeDtypeStruct((B, T, HID), f32),
            jax.ShapeDtypeStruct((D, B * SD), f32),
        ],
        scratch_shapes=[
            pltpu.VMEM((D, NHD), f32),
            pltpu.VMEM((C, 1, NHD), f32),
            pltpu.VMEM((C, 1, NHD), f32),
            pltpu.VMEM((C, 1, NHD), f32),
            pltpu.VMEM((C, 1, NHD), f32),
            pltpu.VMEM((C, HB, D), f32),
            pltpu.VMEM((C, HB, D), f32),
        ],
        compiler_params=pltpu.CompilerParams(
            dimension_semantics=("arbitrary",),
            vmem_limit_bytes=100 * 1024 * 1024,
        ),
        name="gated_delta_state_fused",
    )(hidden_states, S0T, Wp, bb, ba, BD, gwt, wot, nw)

    S_final = soutT.reshape(D, B, H, D).transpose(1, 2, 3, 0)
    return out, S_final
